# Initial kernel scaffold; baseline (speedup 1.0000x reference)
import jax
import jax.numpy as jnp
from jax.experimental import pallas as pl


def kernel(x, segment_ids, emb, W, b):
    raise NotImplementedError("write your pallas kernel here")


# speedup vs baseline: 9.5508x; 9.5508x over previous
import functools

import jax
import jax.numpy as jnp
from jax.experimental import pallas as pl
from jax.experimental.pallas import tpu as pltpu

N_NODES = 100000
HID = 128
NUM_CLASS = 32
NUM_SEG = 256

_R = 2000
_K = N_NODES // _R


def _body(ids_ref, x_ref, emb_ref, W_ref, b_ref, out_ref, acc_ref, cnt_ref):
    k = pl.program_id(0)

    @pl.when(k == 0)
    def _init():
        acc_ref[...] = jnp.zeros_like(acc_ref)
        cnt_ref[...] = jnp.zeros_like(cnt_ref)

    ids = ids_ref[0, 0, :]
    seg_iota = jax.lax.broadcasted_iota(jnp.int32, (NUM_SEG, _R), 0)
    onehot = (ids[None, :] == seg_iota).astype(jnp.float32)
    acc_ref[...] += jax.lax.dot(onehot, x_ref[...],
                                preferred_element_type=jnp.float32)
    cnt_ref[...] += jnp.sum(onehot, axis=1)

    @pl.when(k == _K - 1)
    def _fin():
        counts = jnp.maximum(cnt_ref[...], 1.0)
        mean = acc_ref[...] / counts[:, None]
        cat = jnp.concatenate([mean, emb_ref[...]], axis=1)
        out_ref[...] = jax.lax.dot_general(
            cat, W_ref[...], (((1,), (1,)), ((), ())),
            preferred_element_type=jnp.float32) + b_ref[...]


def kernel(x, segment_ids, emb, W, b):
    ids3 = segment_ids.astype(jnp.int32).reshape(_K, 1, _R)
    b2 = b.reshape(1, NUM_CLASS)
    out = pl.pallas_call(
        _body,
        grid=(_K,),
        in_specs=[
            pl.BlockSpec((1, 1, _R), lambda k: (k, 0, 0)),
            pl.BlockSpec((_R, HID), lambda k: (k, 0)),
            pl.BlockSpec((NUM_SEG, HID), lambda k: (0, 0)),
            pl.BlockSpec((NUM_CLASS, 2 * HID), lambda k: (0, 0)),
            pl.BlockSpec((1, NUM_CLASS), lambda k: (0, 0)),
        ],
        out_specs=pl.BlockSpec((NUM_SEG, NUM_CLASS), lambda k: (0, 0)),
        out_shape=jax.ShapeDtypeStruct((NUM_SEG, NUM_CLASS), jnp.float32),
        scratch_shapes=[
            pltpu.VMEM((NUM_SEG, HID), jnp.float32),
            pltpu.VMEM((NUM_SEG,), jnp.float32),
        ],
    )(ids3, x, emb, W, b2)
    return out
